# baseline (speedup 1.0000x reference)
import jax
import jax.numpy as jnp
from jax import lax
from jax.experimental import pallas as pl
from jax.experimental.pallas import tpu as pltpu
from jax.experimental.pallas import tpu_sc as plsc

Q_TOT = 1024
K_DB = 100000
D_PAD = 32
QB = 256
DBB = 2048
NJ = 49
K_PAD = NJ * DBB
NQ = Q_TOT // QB
K_NN = 5

SC_NC = 2
SC_NS = 16
SC_NW = SC_NC * SC_NS
ROWS_PER_W = Q_TOT // SC_NW


def _topk_body(c_ref, dbt_ref, sel_ref, out_ref, d_ref, v_ref, i_ref):
    j = pl.program_id(1)
    c = c_ref[...]
    dbt = dbt_ref[...]
    q2 = jnp.sum(c * c, axis=1)
    db2 = jnp.sum(dbt * dbt, axis=0)
    mm = jnp.dot(c, dbt, preferred_element_type=jnp.float32)
    d = q2[:, None] - 2.0 * mm + db2[None, :]

    cols = lax.broadcasted_iota(jnp.int32, (QB, DBB), 1) + j * DBB
    d = jnp.where(cols < K_DB, d, jnp.inf)

    @pl.when(j == 0)
    def _init():
        v_ref[...] = jnp.full((K_NN, QB), jnp.inf, jnp.float32)
        i_ref[...] = jnp.zeros((K_NN, QB), jnp.int32)

    d_ref[...] = d

    for _t in range(K_NN):
        dd = d_ref[...]
        m = jnp.min(dd, axis=1)
        cand = jnp.where(dd == m[:, None], cols, jnp.int32(2**31 - 1))
        gi = jnp.min(cand, axis=1)
        d_ref[...] = jnp.where(cols == gi[:, None], jnp.inf, dd)
        ov = [v_ref[s] for s in range(K_NN)]
        oi = [i_ref[s] for s in range(K_NN)]
        for s in range(K_NN):
            ins = m < ov[s]
            if s == 0:
                v_ref[s] = jnp.where(ins, m, ov[s])
                i_ref[s] = jnp.where(ins, gi, oi[s])
            else:
                shift = m < ov[s - 1]
                v_ref[s] = jnp.where(shift, ov[s - 1], jnp.where(ins, m, ov[s]))
                i_ref[s] = jnp.where(shift, oi[s - 1], jnp.where(ins, gi, oi[s]))

    @pl.when(j == NJ - 1)
    def _emit():
        sel = sel_ref[0, 0, :]
        chosen = i_ref[0]
        for t in range(1, K_NN):
            chosen = jnp.where(sel == t, i_ref[t], chosen)
        out_ref[0, 0, :] = chosen


def _topk_call(codes_pad, dbt_pad, sel3):
    return pl.pallas_call(
        _topk_body,
        grid=(NQ, NJ),
        in_specs=[
            pl.BlockSpec((QB, D_PAD), lambda qi, j: (qi, 0)),
            pl.BlockSpec((D_PAD, DBB), lambda qi, j: (0, j)),
            pl.BlockSpec((1, 1, QB), lambda qi, j: (qi, 0, 0)),
        ],
        out_specs=pl.BlockSpec((1, 1, QB), lambda qi, j: (qi, 0, 0)),
        out_shape=jax.ShapeDtypeStruct((NQ, 1, QB), jnp.int32),
        scratch_shapes=[
            pltpu.VMEM((QB, DBB), jnp.float32),
            pltpu.VMEM((K_NN, QB), jnp.float32),
            pltpu.VMEM((K_NN, QB), jnp.int32),
        ],
        compiler_params=pltpu.CompilerParams(
            dimension_semantics=("parallel", "arbitrary")),
    )(codes_pad, dbt_pad, sel3)


def _gather_body(table_hbm, idx_hbm, out_hbm, idx_v, rows_v, sem):
    wid = lax.axis_index("s") * SC_NC + lax.axis_index("c")
    base = wid * ROWS_PER_W
    pltpu.sync_copy(idx_hbm.at[pl.ds(base, ROWS_PER_W)], idx_v)
    pltpu.async_copy(table_hbm.at[idx_v], rows_v, sem).wait()
    pltpu.sync_copy(rows_v, out_hbm.at[pl.ds(base, ROWS_PER_W)])


def _gather_call(table_pad, idx):
    import functools
    mesh = plsc.VectorSubcoreMesh(core_axis_name="c", subcore_axis_name="s")
    k = functools.partial(
        pl.kernel,
        mesh=mesh,
        out_type=jax.ShapeDtypeStruct((Q_TOT, D_PAD), jnp.float32),
        scratch_types=[
            pltpu.VMEM((ROWS_PER_W,), jnp.int32),
            pltpu.VMEM((ROWS_PER_W, D_PAD), jnp.float32),
            pltpu.SemaphoreType.DMA,
        ],
        compiler_params=pltpu.CompilerParams(use_tc_tiling_on_sc=False),
    )(_gather_body)
    return k(table_pad, idx)


def _mlp_body(nb_ref, w1_ref, b1_ref, wu_ref, bu_ref, ws_ref, bs_ref,
              mu_ref, ls_ref):
    nb = nb_ref[...]
    h = jnp.dot(nb, w1_ref[...], preferred_element_type=jnp.float32)
    h = jnp.maximum(h + b1_ref[...], 0.0)
    mu_ref[...] = jnp.dot(h, wu_ref[...],
                          preferred_element_type=jnp.float32) + bu_ref[...]
    ls_ref[...] = jnp.dot(h, ws_ref[...],
                          preferred_element_type=jnp.float32) + bs_ref[...]


def _mlp_call(nb, w1t, b1r, wut, bur, wst, bsr):
    return pl.pallas_call(
        _mlp_body,
        out_shape=(
            jax.ShapeDtypeStruct((Q_TOT, D_PAD), jnp.float32),
            jax.ShapeDtypeStruct((Q_TOT, D_PAD), jnp.float32),
        ),
    )(nb, w1t, b1r, wut, bur, wst, bsr)


def kernel(codes, codes_db, fc1_W, fc1_b, fc2u_W, fc2u_b, fc2s_W, fc2s_b):
    q, d = codes.shape
    kdb = codes_db.shape[0]
    dp = D_PAD - d
    sel = jax.random.randint(jax.random.key(42), (q,), 0, K_NN)

    codes_pad = jnp.pad(codes, ((0, 0), (0, dp)))
    dbt_pad = jnp.pad(codes_db.T, ((0, dp), (0, K_PAD - kdb)))
    table_pad = jnp.pad(codes_db, ((0, 0), (0, dp)))

    idx3 = _topk_call(codes_pad, dbt_pad,
                      sel.astype(jnp.int32).reshape(NQ, 1, QB))
    idx = idx3.reshape(q)

    nb = _gather_call(table_pad, idx)

    w1t = jnp.pad(fc1_W.T, ((0, dp), (0, 0)))
    b1r = fc1_b[None, :]
    wut = jnp.pad(fc2u_W.T, ((0, 0), (0, dp)))
    bur = jnp.pad(fc2u_b, (0, dp))[None, :]
    wst = jnp.pad(fc2s_W.T, ((0, 0), (0, dp)))
    bsr = jnp.pad(fc2s_b, (0, dp))[None, :]

    mu_p, ls_p = _mlp_call(nb, w1t, b1r, wut, bur, wst, bsr)
    return (mu_p[:, :d], ls_p[:, :d])


# speedup vs baseline: 3.9070x; 3.9070x over previous
import functools

import jax
import jax.numpy as jnp
from jax import lax
from jax.experimental import pallas as pl
from jax.experimental.pallas import tpu as pltpu
from jax.experimental.pallas import tpu_sc as plsc

Q_TOT = 1024
K_DB = 100000
D_PAD = 32
QB = 1024
DBB = 2048
NJ = 49
K_PAD = NJ * DBB
NQ = Q_TOT // QB
K_NN = 5

SC_NC = 2
SC_NS = 16
SC_NW = SC_NC * SC_NS
ROWS_PER_W = Q_TOT // SC_NW


def _insert_carry(v_ref, i_ref, m, gi):
    cv = v_ref[...]
    ci = i_ref[...]
    ins = m < cv
    cvp = jnp.concatenate(
        [jnp.full((QB, 1), -jnp.inf, jnp.float32), cv[:, :7]], axis=1)
    insl = m < cvp
    cip = jnp.concatenate([ci[:, :1], ci[:, :7]], axis=1)
    mb = jnp.broadcast_to(m, (QB, 8))
    gb = jnp.broadcast_to(gi, (QB, 8))
    v_ref[...] = jnp.where(insl, cvp, jnp.where(ins, mb, cv))
    i_ref[...] = jnp.where(insl, cip, jnp.where(ins, gb, ci))


def _topk_body(c2_ref, dbt_ref, sel_ref, out_ref, d_ref, v_ref, i_ref,
               mn_ref, flag_ref):
    j = pl.program_id(1)
    c2 = c2_ref[...]
    dbt = dbt_ref[...]
    q2 = 0.25 * jnp.sum(c2 * c2, axis=1, keepdims=True)
    db2 = jnp.sum(dbt * dbt, axis=0, keepdims=True)
    mm2 = jnp.dot(c2, dbt, preferred_element_type=jnp.float32)
    dd = q2 + mm2 + db2

    cols_l = lax.broadcasted_iota(jnp.int32, (QB, DBB), 1).astype(jnp.float32)
    jb = (j * DBB).astype(jnp.float32)

    @pl.when(j == 0)
    def _init():
        v_ref[...] = jnp.full((QB, 8), jnp.inf, jnp.float32)
        i_ref[...] = jnp.zeros((QB, 8), jnp.float32)

    flag_ref[0] = 1

    m = jnp.min(dd, axis=1, keepdims=True)
    cand = jnp.where(dd == m, cols_l, jnp.inf)
    gl = jnp.min(cand, axis=1, keepdims=True)
    x1 = jnp.where(cols_l == gl, jnp.inf, dd)
    _insert_carry(v_ref, i_ref, m, gl + jb)

    m1 = jnp.min(x1, axis=1, keepdims=True)
    cand = jnp.where(x1 == m1, cols_l, jnp.inf)
    gl = jnp.min(cand, axis=1, keepdims=True)
    x2 = jnp.where(cols_l == gl, jnp.inf, x1)
    d_ref[...] = x2
    mn_ref[...] = jnp.min(x2, axis=1, keepdims=True)
    _insert_carry(v_ref, i_ref, m1, gl + jb)

    for t in range(2, K_NN):
        @pl.when(flag_ref[0] == 1)
        def _maybe(t=t):
            mt = mn_ref[...]
            go = jnp.any(mt < v_ref[:, K_NN - 1:K_NN])

            @pl.when(go)
            def _extract():
                ddt = d_ref[...]
                ct = jnp.where(ddt == mt, cols_l, jnp.inf)
                gt = jnp.min(ct, axis=1, keepdims=True)
                if t < K_NN - 1:
                    xn = jnp.where(cols_l == gt, jnp.inf, ddt)
                    d_ref[...] = xn
                    mn_ref[...] = jnp.min(xn, axis=1, keepdims=True)
                _insert_carry(v_ref, i_ref, mt, gt + jb)

            @pl.when(jnp.logical_not(go))
            def _stop():
                flag_ref[0] = 0

    @pl.when(j == NJ - 1)
    def _emit():
        sel = sel_ref[0]
        chosen = i_ref[:, 0:1]
        for t in range(1, K_NN):
            chosen = jnp.where(sel == t, i_ref[:, t:t + 1], chosen)
        out_ref[0] = chosen.astype(jnp.int32)


def _topk_call(codes_pad, dbt_pad, sel3):
    return pl.pallas_call(
        _topk_body,
        grid=(NQ, NJ),
        in_specs=[
            pl.BlockSpec((QB, D_PAD), lambda qi, j: (qi, 0)),
            pl.BlockSpec((D_PAD, DBB), lambda qi, j: (0, j)),
            pl.BlockSpec((1, QB, 1), lambda qi, j: (qi, 0, 0)),
        ],
        out_specs=pl.BlockSpec((1, QB, 1), lambda qi, j: (qi, 0, 0)),
        out_shape=jax.ShapeDtypeStruct((NQ, QB, 1), jnp.int32),
        scratch_shapes=[
            pltpu.VMEM((QB, DBB), jnp.float32),
            pltpu.VMEM((QB, 8), jnp.float32),
            pltpu.VMEM((QB, 8), jnp.float32),
            pltpu.VMEM((QB, 1), jnp.float32),
            pltpu.SMEM((1,), jnp.int32),
        ],
        compiler_params=pltpu.CompilerParams(
            dimension_semantics=("parallel", "arbitrary")),
    )(codes_pad, dbt_pad, sel3)


def _gather_body(table_hbm, idx_hbm, out_hbm, idx_v, rows_v, sem):
    wid = lax.axis_index("s") * SC_NC + lax.axis_index("c")
    base = wid * ROWS_PER_W
    pltpu.sync_copy(idx_hbm.at[pl.ds(base, ROWS_PER_W)], idx_v)
    pltpu.async_copy(table_hbm.at[idx_v], rows_v, sem).wait()
    pltpu.sync_copy(rows_v, out_hbm.at[pl.ds(base, ROWS_PER_W)])


def _gather_call(table, idx, d):
    mesh = plsc.VectorSubcoreMesh(core_axis_name="c", subcore_axis_name="s")
    k = functools.partial(
        pl.kernel,
        mesh=mesh,
        out_type=jax.ShapeDtypeStruct((Q_TOT, d), jnp.float32),
        scratch_types=[
            pltpu.VMEM((ROWS_PER_W,), jnp.int32),
            pltpu.VMEM((ROWS_PER_W, d), jnp.float32),
            pltpu.SemaphoreType.DMA,
        ],
        compiler_params=pltpu.CompilerParams(use_tc_tiling_on_sc=False),
    )(_gather_body)
    return k(table, idx)


def _mlp_body(nb_ref, w1_ref, b1_ref, wu_ref, bu_ref, ws_ref, bs_ref,
              mu_ref, ls_ref):
    nb = nb_ref[...]
    h = jnp.dot(nb, w1_ref[...], preferred_element_type=jnp.float32)
    h = jnp.maximum(h + b1_ref[...], 0.0)
    mu_ref[...] = jnp.dot(h, wu_ref[...],
                          preferred_element_type=jnp.float32) + bu_ref[...]
    ls_ref[...] = jnp.dot(h, ws_ref[...],
                          preferred_element_type=jnp.float32) + bs_ref[...]


def _mlp_call(nb, w1t, b1r, wut, bur, wst, bsr):
    return pl.pallas_call(
        _mlp_body,
        out_shape=(
            jax.ShapeDtypeStruct((Q_TOT, D_PAD), jnp.float32),
            jax.ShapeDtypeStruct((Q_TOT, D_PAD), jnp.float32),
        ),
    )(nb, w1t, b1r, wut, bur, wst, bsr)


def kernel(codes, codes_db, fc1_W, fc1_b, fc2u_W, fc2u_b, fc2s_W, fc2s_b):
    q, d = codes.shape
    kdb = codes_db.shape[0]
    dp = D_PAD - d
    sel = jax.random.randint(jax.random.key(42), (q,), 0, K_NN)

    codes2_pad = jnp.pad(-2.0 * codes, ((0, 0), (0, dp)))
    dbt_pad = jnp.pad(
        jnp.pad(codes_db.T, ((0, 0), (0, K_PAD - kdb)),
                constant_values=1e19),
        ((0, dp), (0, 0)))
    table_pad = jnp.pad(codes_db, ((0, 0), (0, dp)))

    idx3 = _topk_call(codes2_pad, dbt_pad,
                      sel.astype(jnp.int32).reshape(NQ, QB, 1))
    idx = idx3.reshape(q)

    nb = _gather_call(table_pad, idx, D_PAD)

    w1t = jnp.pad(fc1_W.T, ((0, dp), (0, 0)))
    b1r = fc1_b[None, :]
    wut = jnp.pad(fc2u_W.T, ((0, 0), (0, dp)))
    bur = jnp.pad(fc2u_b, (0, dp))[None, :]
    wst = jnp.pad(fc2s_W.T, ((0, 0), (0, dp)))
    bsr = jnp.pad(fc2s_b, (0, dp))[None, :]

    mu_p, ls_p = _mlp_call(nb, w1t, b1r, wut, bur, wst, bsr)
    return (mu_p[:, :d], ls_p[:, :d])
